# Initial kernel scaffold; baseline (speedup 1.0000x reference)
import jax
import jax.numpy as jnp
from jax.experimental import pallas as pl


def kernel(x, edge_attr, params, edge_index):
    raise NotImplementedError("write your pallas kernel here")


# speedup vs baseline: 2.5167x; 2.5167x over previous
import functools

import jax
import jax.numpy as jnp
from jax import lax
from jax.experimental import pallas as pl
from jax.experimental.pallas import tpu as pltpu
from jax.experimental.pallas import tpu_sc as plsc

_NC = 2
_NS = 16
_CH = 80


def _gine_agg(x, ea, src, dst):
    N, C = x.shape
    E = src.shape[0]
    EW = E // (_NC * _NS)
    nchunk = EW // _CH
    rows_w = (N // _NS) // 8 * 8
    rem = N - _NS * rows_w
    zeros = jnp.zeros((N, C), jnp.float32)
    mesh = plsc.VectorSubcoreMesh(core_axis_name="c", subcore_axis_name="s")

    @functools.partial(
        pl.kernel,
        mesh=mesh,
        out_type=jax.ShapeDtypeStruct((_NC * N, C), jnp.float32),
        scratch_types=[
            pltpu.VMEM((_CH,), jnp.int32),
            pltpu.VMEM((_CH,), jnp.int32),
            pltpu.VMEM((_CH, C), jnp.float32),
            pltpu.VMEM((_CH, C), jnp.float32),
            pltpu.VMEM_SHARED((N, C), jnp.float32),
            pltpu.SemaphoreType.DMA,
        ],
    )
    def k(x_hbm, ea_hbm, src_hbm, dst_hbm, zero_hbm, out_hbm,
          srcv, dstv, xg, eav, aggsh, sem):
        cid = lax.axis_index("c")
        sid = lax.axis_index("s")
        r0 = sid * rows_w
        pltpu.sync_copy(zero_hbm.at[pl.ds(r0, rows_w)],
                        aggsh.at[pl.ds(r0, rows_w)])

        @pl.when(sid == 0)
        def _():
            pltpu.sync_copy(zero_hbm.at[pl.ds(_NS * rows_w, rem)],
                            aggsh.at[pl.ds(_NS * rows_w, rem)])

        plsc.subcore_barrier()

        base = cid * (E // _NC) + sid * EW

        def chunk(i, carry):
            off = base + i * _CH
            pltpu.sync_copy(src_hbm.at[pl.ds(off, _CH)], srcv)
            pltpu.sync_copy(dst_hbm.at[pl.ds(off, _CH)], dstv)
            pltpu.async_copy(x_hbm.at[srcv], xg, sem).wait()
            pltpu.sync_copy(ea_hbm.at[pl.ds(off, _CH)], eav)

            def row(r, c2):
                for cc in range(C // 16):
                    sl = pl.ds(cc * 16, 16)
                    xg[r, sl] = jnp.maximum(xg[r, sl] + eav[r, sl], 0.0)
                return c2

            lax.fori_loop(0, _CH, row, 0)
            pltpu.sync_copy(xg, aggsh.at[dstv], add=True)
            return carry

        lax.fori_loop(0, nchunk, chunk, 0)
        plsc.subcore_barrier()
        pltpu.sync_copy(aggsh.at[pl.ds(r0, rows_w)],
                        out_hbm.at[pl.ds(cid * N + r0, rows_w)])

        @pl.when(sid == 0)
        def _():
            pltpu.sync_copy(aggsh.at[pl.ds(_NS * rows_w, rem)],
                            out_hbm.at[pl.ds(cid * N + _NS * rows_w, rem)])

    return k(x, ea, src, dst, zeros).reshape(_NC, N, C)


def _ea_proj_body(a_ref, w1_ref, b1_ref, w2_ref, b2_ref, o1_ref, o2_ref):
    a = a_ref[...]
    o1_ref[...] = jnp.dot(a, w1_ref[...],
                          preferred_element_type=jnp.float32) + b1_ref[...]
    o2_ref[...] = jnp.dot(a, w2_ref[...],
                          preferred_element_type=jnp.float32) + b2_ref[...]


def _ea_proj(edge_attr, W1, b1, W2, b2):
    E, K = edge_attr.shape
    C = W1.shape[1]
    BE = 8000
    return pl.pallas_call(
        _ea_proj_body,
        grid=(E // BE,),
        in_specs=[
            pl.BlockSpec((BE, K), lambda i: (i, 0)),
            pl.BlockSpec((K, C), lambda i: (0, 0)),
            pl.BlockSpec((1, C), lambda i: (0, 0)),
            pl.BlockSpec((K, C), lambda i: (0, 0)),
            pl.BlockSpec((1, C), lambda i: (0, 0)),
        ],
        out_specs=[pl.BlockSpec((BE, C), lambda i: (i, 0))] * 2,
        out_shape=[jax.ShapeDtypeStruct((E, C), jnp.float32)] * 2,
    )(edge_attr, W1, b1.reshape(1, C), W2, b2.reshape(1, C))


def _qkv_body(x_ref, wq, bq, wk, bk, wv, bv, q_ref, k_ref, v_ref):
    xv = x_ref[...]
    q_ref[...] = jnp.dot(xv, wq[...], preferred_element_type=jnp.float32) + bq[...]
    k_ref[...] = jnp.dot(xv, wk[...], preferred_element_type=jnp.float32) + bk[...]
    v_ref[...] = jnp.dot(xv, wv[...], preferred_element_type=jnp.float32) + bv[...]


def _qkv(x, p):
    N, C = x.shape
    return pl.pallas_call(
        _qkv_body,
        out_shape=[jax.ShapeDtypeStruct((N, C), jnp.float32)] * 3,
    )(x, p['Wq'], p['bq'].reshape(1, C), p['Wk'], p['bk'].reshape(1, C),
      p['Wv'], p['bv'].reshape(1, C))


def _attn_body(heads, scale, q_ref, k_ref, v_ref, o_ref):
    C = q_ref.shape[1]
    dh = C // heads
    q = q_ref[...]
    kk = k_ref[...]
    vv = v_ref[...]
    outs = []
    for h in range(heads):
        sl = slice(h * dh, (h + 1) * dh)
        s = lax.dot_general(q[:, sl], kk[:, sl], (((1,), (1,)), ((), ())),
                            preferred_element_type=jnp.float32) * scale
        m = jnp.max(s, axis=1, keepdims=True)
        p = jnp.exp(s - m)
        l = jnp.sum(p, axis=1, keepdims=True)
        outs.append(jnp.dot(p, vv[:, sl],
                            preferred_element_type=jnp.float32) / l)
    o_ref[...] = jnp.concatenate(outs, axis=1) if heads > 1 else outs[0]


def _attention(q, k, v, heads):
    N, C = q.shape
    dh = C // heads
    BQ = 400
    body = functools.partial(_attn_body, heads, float(dh) ** -0.5)
    return pl.pallas_call(
        body,
        grid=(N // BQ,),
        in_specs=[
            pl.BlockSpec((BQ, C), lambda qb: (qb, 0)),
            pl.BlockSpec((N, C), lambda qb: (0, 0)),
            pl.BlockSpec((N, C), lambda qb: (0, 0)),
        ],
        out_specs=pl.BlockSpec((BQ, C), lambda qb: (qb, 0)),
        out_shape=jax.ShapeDtypeStruct((N, C), jnp.float32),
    )(q, k, v)


def _bn_in(t, g, b, eps=1e-5):
    mu = jnp.mean(t, axis=0, keepdims=True)
    d = t - mu
    var = jnp.mean(d * d, axis=0, keepdims=True)
    return d * lax.rsqrt(var + eps) * g + b


def _post1_body(x_ref, agg_ref, attn_ref, w1, b1, w2, b2, wo, bo,
                n1g, n1b, n2g, n2b, out_ref):
    xv = x_ref[...]
    h = xv + agg_ref[0] + agg_ref[1]
    hid = jax.nn.relu(jnp.dot(h, w1[...],
                              preferred_element_type=jnp.float32) + b1[...])
    gm = jnp.dot(hid, w2[...], preferred_element_type=jnp.float32) + b2[...]
    bnh = _bn_in(gm + xv, n1g[...], n1b[...])
    ao = jnp.dot(attn_ref[...], wo[...],
                 preferred_element_type=jnp.float32) + bo[...]
    bnha = _bn_in(ao + xv, n2g[...], n2b[...])
    out_ref[...] = bnh + bnha


def _post1(x, agg, attn, p):
    N, C = x.shape
    r = lambda a: a.reshape(1, C)
    return pl.pallas_call(
        _post1_body,
        out_shape=jax.ShapeDtypeStruct((N, C), jnp.float32),
    )(x, agg, attn, p['W1'], r(p['b1']), p['W2'], r(p['b2']),
      p['Wo'], r(p['bo']), r(p['n1_g']), r(p['n1_b']),
      r(p['n2_g']), r(p['n2_b']))


def _post2_body(final, z_ref, mw1, mb1, mw2, mb2, n3g, n3b,
                bng, bnb, linw, linb, bn2g, bn2b, out_ref):
    z = z_ref[...]
    hid = jax.nn.relu(jnp.dot(z, mw1[...],
                              preferred_element_type=jnp.float32) + mb1[...])
    z2 = z + jnp.dot(hid, mw2[...],
                     preferred_element_type=jnp.float32) + mb2[...]
    z3 = _bn_in(z2, n3g[...], n3b[...])
    y = jax.nn.relu(_bn_in(z3, bng[...], bnb[...]))
    y2 = jnp.dot(y, linw[...], preferred_element_type=jnp.float32) + linb[...]
    if final:
        out_ref[...] = y2
    else:
        out_ref[...] = jax.nn.relu(_bn_in(y2, bn2g[...], bn2b[...]))


def _post2(z, gp, n_g, n_b, linW, linb, bn2_g, bn2_b, final):
    N, C = z.shape
    CO = linW.shape[1]
    if final:
        bn2_g = jnp.zeros((CO,), jnp.float32)
        bn2_b = bn2_g
    r = lambda a: a.reshape(1, -1)
    body = functools.partial(_post2_body, final)
    return pl.pallas_call(
        body,
        out_shape=jax.ShapeDtypeStruct((N, CO), jnp.float32),
    )(z, gp['mW1'], r(gp['mb1']), gp['mW2'], r(gp['mb2']),
      r(gp['n3_g']), r(gp['n3_b']), r(n_g), r(n_b),
      linW, r(linb), r(bn2_g), r(bn2_b))


def _gps_layer(x, ea, src, dst, p, heads):
    q, k, v = _qkv(x, p)
    agg = _gine_agg(x, ea, src, dst)
    attn = _attention(q, k, v, heads)
    return _post1(x, agg, attn, p)


@jax.jit
def kernel(x, edge_attr, params, edge_index):
    src = edge_index[0]
    dst = edge_index[1]
    p1 = params['gps1']
    p2 = params['gps2']
    ea1, ea2 = _ea_proj(edge_attr, p1['We'], p1['be'], p2['We'], p2['be'])

    z = _gps_layer(x, ea1, src, dst, p1, 2)
    h = _post2(z, p1, params['bn1_g'], params['bn1_b'],
               params['lin1_W'], params['lin1_b'],
               params['bn2_g'], params['bn2_b'], final=False)

    z = _gps_layer(h, ea2, src, dst, p2, 1)
    out = _post2(z, p2, params['bn2_g'], params['bn2_b'],
                 params['lin2_W'], params['lin2_b'],
                 None, None, final=True)
    return out


# speedup vs baseline: 3.3142x; 1.3169x over previous
import functools

import jax
import jax.numpy as jnp
from jax import lax
from jax.experimental import pallas as pl
from jax.experimental.pallas import tpu as pltpu
from jax.experimental.pallas import tpu_sc as plsc

_NC = 2
_NS = 16
_CH = 40


def _gine_agg(x, ea, src, dst):
    N, C = x.shape
    E = src.shape[0]
    EW = E // (_NC * _NS)
    nchunk = EW // _CH
    rows_w = (N // _NS) // 8 * 8
    rem = N - _NS * rows_w
    zeros = jnp.zeros((N, C), jnp.float32)
    mesh = plsc.VectorSubcoreMesh(core_axis_name="c", subcore_axis_name="s")

    idxv = lambda: pltpu.VMEM((_CH,), jnp.int32)
    rowv = lambda: pltpu.VMEM((_CH, C), jnp.float32)

    @functools.partial(
        pl.kernel,
        mesh=mesh,
        out_type=jax.ShapeDtypeStruct((_NC * N, C), jnp.float32),
        scratch_types=[
            idxv(), idxv(), idxv(), idxv(), idxv(), idxv(),
            rowv(), rowv(), rowv(), rowv(), rowv(), rowv(),
            pltpu.VMEM_SHARED((N, C), jnp.float32),
            pltpu.SemaphoreType.DMA, pltpu.SemaphoreType.DMA,
            pltpu.SemaphoreType.DMA, pltpu.SemaphoreType.DMA,
            pltpu.SemaphoreType.DMA, pltpu.SemaphoreType.DMA,
        ],
    )
    def k(x_hbm, ea_hbm, src_hbm, dst_hbm, zero_hbm, out_hbm,
          src0, src1, src2, dst0, dst1, dst2,
          xg0, xg1, xg2, ea0, ea1, ea2, aggsh,
          semi0, semi1, semi2, semg0, semg1, semg2):
        cid = lax.axis_index("c")
        sid = lax.axis_index("s")
        r0 = sid * rows_w
        base = cid * (E // _NC) + sid * EW
        slot = ((src0, dst0, xg0, ea0, semi0, semg0),
                (src1, dst1, xg1, ea1, semi1, semg1),
                (src2, dst2, xg2, ea2, semi2, semg2))

        pltpu.sync_copy(zero_hbm.at[pl.ds(r0, rows_w)],
                        aggsh.at[pl.ds(r0, rows_w)])

        @pl.when(sid == 0)
        def _():
            pltpu.sync_copy(zero_hbm.at[pl.ds(_NS * rows_w, rem)],
                            aggsh.at[pl.ds(_NS * rows_w, rem)])

        plsc.subcore_barrier()

        def start_idx(i, b):
            srcv, dstv, _, eav, semi, _ = slot[b]
            off = base + i * _CH
            pltpu.async_copy(src_hbm.at[pl.ds(off, _CH)], srcv, semi)
            pltpu.async_copy(dst_hbm.at[pl.ds(off, _CH)], dstv, semi)
            pltpu.async_copy(ea_hbm.at[pl.ds(off, _CH)], eav, semi)

        def gather(i, b):
            srcv, dstv, xg, eav, semi, semg = slot[b]
            off = base + i * _CH
            pltpu.make_async_copy(src_hbm.at[pl.ds(off, _CH)], srcv,
                                  semi).wait()
            pltpu.make_async_copy(dst_hbm.at[pl.ds(off, _CH)], dstv,
                                  semi).wait()
            pltpu.make_async_copy(ea_hbm.at[pl.ds(off, _CH)], eav,
                                  semi).wait()
            pltpu.async_copy(x_hbm.at[srcv], xg, semg)

        def work(b):
            srcv, dstv, xg, eav, _, semg = slot[b]
            pltpu.make_async_copy(x_hbm.at[srcv], xg, semg).wait()

            def row(r, c2):
                for cc in range(C // 16):
                    sl = pl.ds(cc * 16, 16)
                    xg[r, sl] = jnp.maximum(xg[r, sl] + eav[r, sl], 0.0)
                return c2

            lax.fori_loop(0, _CH, row, 0)
            pltpu.sync_copy(xg, aggsh.at[dstv], add=True)

        def step(i, a, b, c):
            start_idx(i + 2, c)
            gather(i + 1, b)
            work(a)

        start_idx(0, 0)
        start_idx(1, 1)
        gather(0, 0)

        def triple(t, carry):
            i = 3 * t
            step(i, 0, 1, 2)
            step(i + 1, 1, 2, 0)
            step(i + 2, 2, 0, 1)
            return carry

        nfull = nchunk - 4
        lax.fori_loop(0, nfull // 3, triple, 0)
        a, b, c = nfull % 3, (nfull + 1) % 3, (nfull + 2) % 3
        step(nfull, a, b, c)
        step(nfull + 1, b, c, a)
        gather(nchunk - 1, a)
        work(c)
        work(a)

        plsc.subcore_barrier()
        pltpu.sync_copy(aggsh.at[pl.ds(r0, rows_w)],
                        out_hbm.at[pl.ds(cid * N + r0, rows_w)])

        @pl.when(sid == 0)
        def _():
            pltpu.sync_copy(aggsh.at[pl.ds(_NS * rows_w, rem)],
                            out_hbm.at[pl.ds(cid * N + _NS * rows_w, rem)])

    return k(x, ea, src, dst, zeros).reshape(_NC, N, C)


def _ea_proj_body(a_ref, w1_ref, b1_ref, w2_ref, b2_ref, o1_ref, o2_ref):
    a = a_ref[...]
    o1_ref[...] = jnp.dot(a, w1_ref[...],
                          preferred_element_type=jnp.float32) + b1_ref[...]
    o2_ref[...] = jnp.dot(a, w2_ref[...],
                          preferred_element_type=jnp.float32) + b2_ref[...]


def _ea_proj(edge_attr, W1, b1, W2, b2):
    E, K = edge_attr.shape
    C = W1.shape[1]
    BE = 8000
    return pl.pallas_call(
        _ea_proj_body,
        grid=(E // BE,),
        in_specs=[
            pl.BlockSpec((BE, K), lambda i: (i, 0)),
            pl.BlockSpec((K, C), lambda i: (0, 0)),
            pl.BlockSpec((1, C), lambda i: (0, 0)),
            pl.BlockSpec((K, C), lambda i: (0, 0)),
            pl.BlockSpec((1, C), lambda i: (0, 0)),
        ],
        out_specs=[pl.BlockSpec((BE, C), lambda i: (i, 0))] * 2,
        out_shape=[jax.ShapeDtypeStruct((E, C), jnp.float32)] * 2,
    )(edge_attr, W1, b1.reshape(1, C), W2, b2.reshape(1, C))


def _qkv_body(x_ref, wq, bq, wk, bk, wv, bv, q_ref, k_ref, v_ref):
    xv = x_ref[...]
    q_ref[...] = jnp.dot(xv, wq[...], preferred_element_type=jnp.float32) + bq[...]
    k_ref[...] = jnp.dot(xv, wk[...], preferred_element_type=jnp.float32) + bk[...]
    v_ref[...] = jnp.dot(xv, wv[...], preferred_element_type=jnp.float32) + bv[...]


def _qkv(x, p):
    N, C = x.shape
    return pl.pallas_call(
        _qkv_body,
        out_shape=[jax.ShapeDtypeStruct((N, C), jnp.float32)] * 3,
    )(x, p['Wq'], p['bq'].reshape(1, C), p['Wk'], p['bk'].reshape(1, C),
      p['Wv'], p['bv'].reshape(1, C))


def _attn_body(heads, scale, q_ref, k_ref, v_ref, o_ref):
    C = q_ref.shape[1]
    dh = C // heads
    q = q_ref[...]
    kk = k_ref[...]
    vv = v_ref[...]
    outs = []
    for h in range(heads):
        sl = slice(h * dh, (h + 1) * dh)
        s = lax.dot_general(q[:, sl], kk[:, sl], (((1,), (1,)), ((), ())),
                            preferred_element_type=jnp.float32) * scale
        m = jnp.max(s, axis=1, keepdims=True)
        p = jnp.exp(s - m)
        l = jnp.sum(p, axis=1, keepdims=True)
        outs.append(jnp.dot(p, vv[:, sl],
                            preferred_element_type=jnp.float32) / l)
    o_ref[...] = jnp.concatenate(outs, axis=1) if heads > 1 else outs[0]


def _attention(q, k, v, heads):
    N, C = q.shape
    dh = C // heads
    BQ = 400
    body = functools.partial(_attn_body, heads, float(dh) ** -0.5)
    return pl.pallas_call(
        body,
        grid=(N // BQ,),
        in_specs=[
            pl.BlockSpec((BQ, C), lambda qb: (qb, 0)),
            pl.BlockSpec((N, C), lambda qb: (0, 0)),
            pl.BlockSpec((N, C), lambda qb: (0, 0)),
        ],
        out_specs=pl.BlockSpec((BQ, C), lambda qb: (qb, 0)),
        out_shape=jax.ShapeDtypeStruct((N, C), jnp.float32),
    )(q, k, v)


def _bn_in(t, g, b, eps=1e-5):
    mu = jnp.mean(t, axis=0, keepdims=True)
    d = t - mu
    var = jnp.mean(d * d, axis=0, keepdims=True)
    return d * lax.rsqrt(var + eps) * g + b


def _post1_body(x_ref, agg_ref, attn_ref, w1, b1, w2, b2, wo, bo,
                n1g, n1b, n2g, n2b, out_ref):
    xv = x_ref[...]
    h = xv + agg_ref[0] + agg_ref[1]
    hid = jax.nn.relu(jnp.dot(h, w1[...],
                              preferred_element_type=jnp.float32) + b1[...])
    gm = jnp.dot(hid, w2[...], preferred_element_type=jnp.float32) + b2[...]
    bnh = _bn_in(gm + xv, n1g[...], n1b[...])
    ao = jnp.dot(attn_ref[...], wo[...],
                 preferred_element_type=jnp.float32) + bo[...]
    bnha = _bn_in(ao + xv, n2g[...], n2b[...])
    out_ref[...] = bnh + bnha


def _post1(x, agg, attn, p):
    N, C = x.shape
    r = lambda a: a.reshape(1, C)
    return pl.pallas_call(
        _post1_body,
        out_shape=jax.ShapeDtypeStruct((N, C), jnp.float32),
    )(x, agg, attn, p['W1'], r(p['b1']), p['W2'], r(p['b2']),
      p['Wo'], r(p['bo']), r(p['n1_g']), r(p['n1_b']),
      r(p['n2_g']), r(p['n2_b']))


def _post2_body(final, z_ref, mw1, mb1, mw2, mb2, n3g, n3b,
                bng, bnb, linw, linb, bn2g, bn2b, out_ref):
    z = z_ref[...]
    hid = jax.nn.relu(jnp.dot(z, mw1[...],
                              preferred_element_type=jnp.float32) + mb1[...])
    z2 = z + jnp.dot(hid, mw2[...],
                     preferred_element_type=jnp.float32) + mb2[...]
    z3 = _bn_in(z2, n3g[...], n3b[...])
    y = jax.nn.relu(_bn_in(z3, bng[...], bnb[...]))
    y2 = jnp.dot(y, linw[...], preferred_element_type=jnp.float32) + linb[...]
    if final:
        out_ref[...] = y2
    else:
        out_ref[...] = jax.nn.relu(_bn_in(y2, bn2g[...], bn2b[...]))


def _post2(z, gp, n_g, n_b, linW, linb, bn2_g, bn2_b, final):
    N, C = z.shape
    CO = linW.shape[1]
    if final:
        bn2_g = jnp.zeros((CO,), jnp.float32)
        bn2_b = bn2_g
    r = lambda a: a.reshape(1, -1)
    body = functools.partial(_post2_body, final)
    return pl.pallas_call(
        body,
        out_shape=jax.ShapeDtypeStruct((N, CO), jnp.float32),
    )(z, gp['mW1'], r(gp['mb1']), gp['mW2'], r(gp['mb2']),
      r(gp['n3_g']), r(gp['n3_b']), r(n_g), r(n_b),
      linW, r(linb), r(bn2_g), r(bn2_b))


def _gps_layer(x, ea, src, dst, p, heads):
    q, k, v = _qkv(x, p)
    agg = _gine_agg(x, ea, src, dst)
    attn = _attention(q, k, v, heads)
    return _post1(x, agg, attn, p)


@jax.jit
def kernel(x, edge_attr, params, edge_index):
    src = edge_index[0]
    dst = edge_index[1]
    p1 = params['gps1']
    p2 = params['gps2']
    ea1, ea2 = _ea_proj(edge_attr, p1['We'], p1['be'], p2['We'], p2['be'])

    z = _gps_layer(x, ea1, src, dst, p1, 2)
    h = _post2(z, p1, params['bn1_g'], params['bn1_b'],
               params['lin1_W'], params['lin1_b'],
               params['bn2_g'], params['bn2_b'], final=False)

    z = _gps_layer(h, ea2, src, dst, p2, 1)
    out = _post2(z, p2, params['bn2_g'], params['bn2_b'],
                 params['lin2_W'], params['lin2_b'],
                 None, None, final=True)
    return out


# speedup vs baseline: 3.4704x; 1.0471x over previous
import functools

import jax
import jax.numpy as jnp
from jax import lax
from jax.experimental import pallas as pl
from jax.experimental.pallas import tpu as pltpu
from jax.experimental.pallas import tpu_sc as plsc

_NC = 2
_NS = 16
_CH = 40


def _gine_agg(x, ea, src, dst):
    N, C = x.shape
    E = src.shape[0]
    EW = E // (_NC * _NS)
    nchunk = EW // _CH
    rows_w = (N // _NS) // 8 * 8
    rem = N - _NS * rows_w
    zeros = jnp.zeros((N, C), jnp.float32)
    mesh = plsc.VectorSubcoreMesh(core_axis_name="c", subcore_axis_name="s")

    idxv = lambda: pltpu.VMEM((_CH,), jnp.int32)
    rowv = lambda: pltpu.VMEM((_CH, C), jnp.float32)

    @functools.partial(
        pl.kernel,
        mesh=mesh,
        out_type=jax.ShapeDtypeStruct((_NC * N, C), jnp.float32),
        scratch_types=[
            idxv(), idxv(), idxv(), idxv(), idxv(), idxv(),
            rowv(), rowv(), rowv(), rowv(), rowv(), rowv(),
            pltpu.VMEM_SHARED((N, C), jnp.float32),
            pltpu.SemaphoreType.DMA, pltpu.SemaphoreType.DMA,
            pltpu.SemaphoreType.DMA, pltpu.SemaphoreType.DMA,
            pltpu.SemaphoreType.DMA, pltpu.SemaphoreType.DMA,
        ],
    )
    def k(x_hbm, ea_hbm, src_hbm, dst_hbm, zero_hbm, out_hbm,
          src0, src1, src2, dst0, dst1, dst2,
          xg0, xg1, xg2, ea0, ea1, ea2, aggsh,
          semi0, semi1, semi2, semg0, semg1, semg2):
        cid = lax.axis_index("c")
        sid = lax.axis_index("s")
        r0 = sid * rows_w
        base = cid * (E // _NC) + sid * EW
        slot = ((src0, dst0, xg0, ea0, semi0, semg0),
                (src1, dst1, xg1, ea1, semi1, semg1),
                (src2, dst2, xg2, ea2, semi2, semg2))

        pltpu.sync_copy(zero_hbm.at[pl.ds(r0, rows_w)],
                        aggsh.at[pl.ds(r0, rows_w)])

        @pl.when(sid == 0)
        def _():
            pltpu.sync_copy(zero_hbm.at[pl.ds(_NS * rows_w, rem)],
                            aggsh.at[pl.ds(_NS * rows_w, rem)])

        plsc.subcore_barrier()

        def start_idx(i, b):
            srcv, dstv, _, eav, semi, _ = slot[b]
            off = base + i * _CH
            pltpu.async_copy(src_hbm.at[pl.ds(off, _CH)], srcv, semi)
            pltpu.async_copy(dst_hbm.at[pl.ds(off, _CH)], dstv, semi)
            pltpu.async_copy(ea_hbm.at[pl.ds(off, _CH)], eav, semi)

        def gather(i, b):
            srcv, dstv, xg, eav, semi, semg = slot[b]
            off = base + i * _CH
            pltpu.make_async_copy(src_hbm.at[pl.ds(off, _CH)], srcv,
                                  semi).wait()
            pltpu.make_async_copy(dst_hbm.at[pl.ds(off, _CH)], dstv,
                                  semi).wait()
            pltpu.make_async_copy(ea_hbm.at[pl.ds(off, _CH)], eav,
                                  semi).wait()
            pltpu.async_copy(x_hbm.at[srcv], xg, semg)

        def work(b):
            srcv, dstv, xg, eav, _, semg = slot[b]
            pltpu.make_async_copy(x_hbm.at[srcv], xg, semg).wait()

            def row(r, c2):
                for cc in range(C // 16):
                    sl = pl.ds(cc * 16, 16)
                    xg[r, sl] = jnp.maximum(xg[r, sl] + eav[r, sl], 0.0)
                return c2

            lax.fori_loop(0, _CH, row, 0)
            pltpu.sync_copy(xg, aggsh.at[dstv], add=True)

        def step(i, a, b, c):
            start_idx(i + 2, c)
            gather(i + 1, b)
            work(a)

        start_idx(0, 0)
        start_idx(1, 1)
        gather(0, 0)

        def triple(t, carry):
            i = 3 * t
            step(i, 0, 1, 2)
            step(i + 1, 1, 2, 0)
            step(i + 2, 2, 0, 1)
            return carry

        nfull = nchunk - 4
        lax.fori_loop(0, nfull // 3, triple, 0)
        a, b, c = nfull % 3, (nfull + 1) % 3, (nfull + 2) % 3
        step(nfull, a, b, c)
        step(nfull + 1, b, c, a)
        gather(nchunk - 1, a)
        work(c)
        work(a)

        plsc.subcore_barrier()
        pltpu.sync_copy(aggsh.at[pl.ds(r0, rows_w)],
                        out_hbm.at[pl.ds(cid * N + r0, rows_w)])

        @pl.when(sid == 0)
        def _():
            pltpu.sync_copy(aggsh.at[pl.ds(_NS * rows_w, rem)],
                            out_hbm.at[pl.ds(cid * N + _NS * rows_w, rem)])

    return k(x, ea, src, dst, zeros).reshape(_NC, N, C)


def _ea_proj_body(a_ref, w1_ref, b1_ref, w2_ref, b2_ref, o1_ref, o2_ref):
    a = a_ref[...]
    o1_ref[...] = jnp.dot(a, w1_ref[...],
                          preferred_element_type=jnp.float32) + b1_ref[...]
    o2_ref[...] = jnp.dot(a, w2_ref[...],
                          preferred_element_type=jnp.float32) + b2_ref[...]


def _ea_proj(edge_attr, W1, b1, W2, b2):
    E, K = edge_attr.shape
    C = W1.shape[1]
    BE = 8000
    return pl.pallas_call(
        _ea_proj_body,
        grid=(E // BE,),
        in_specs=[
            pl.BlockSpec((BE, K), lambda i: (i, 0)),
            pl.BlockSpec((K, C), lambda i: (0, 0)),
            pl.BlockSpec((1, C), lambda i: (0, 0)),
            pl.BlockSpec((K, C), lambda i: (0, 0)),
            pl.BlockSpec((1, C), lambda i: (0, 0)),
        ],
        out_specs=[pl.BlockSpec((BE, C), lambda i: (i, 0))] * 2,
        out_shape=[jax.ShapeDtypeStruct((E, C), jnp.float32)] * 2,
    )(edge_attr, W1, b1.reshape(1, C), W2, b2.reshape(1, C))


def _qkv_body(scale, x_ref, wq, bq, wk, bk, wv, bv, q_ref, k_ref, v_ref):
    xv = x_ref[...]
    q = jnp.dot(xv, wq[...], preferred_element_type=jnp.float32) + bq[...]
    q_ref[...] = (q * scale).astype(jnp.bfloat16)
    k = jnp.dot(xv, wk[...], preferred_element_type=jnp.float32) + bk[...]
    k_ref[...] = k.astype(jnp.bfloat16)
    v = jnp.dot(xv, wv[...], preferred_element_type=jnp.float32) + bv[...]
    v_ref[...] = v.astype(jnp.bfloat16)


def _qkv(x, p, heads):
    N, C = x.shape
    scale = float(C // heads) ** -0.5
    return pl.pallas_call(
        functools.partial(_qkv_body, scale),
        out_shape=[jax.ShapeDtypeStruct((N, C), jnp.bfloat16)] * 3,
    )(x, p['Wq'], p['bq'].reshape(1, C), p['Wk'], p['bk'].reshape(1, C),
      p['Wv'], p['bv'].reshape(1, C))


def _attn_body(heads, q_ref, k_ref, v_ref, o_ref):
    C = q_ref.shape[1]
    dh = C // heads
    q = q_ref[...]
    kk = k_ref[...]
    vv = v_ref[...]
    outs = []
    for h in range(heads):
        sl = slice(h * dh, (h + 1) * dh)
        s = lax.dot_general(q[:, sl], kk[:, sl], (((1,), (1,)), ((), ())),
                            preferred_element_type=jnp.float32)
        m = jnp.max(s, axis=1, keepdims=True)
        p = jnp.exp(s - m)
        l = jnp.sum(p, axis=1, keepdims=True)
        outs.append(jnp.dot(p.astype(jnp.bfloat16), vv[:, sl],
                            preferred_element_type=jnp.float32) / l)
    o_ref[...] = jnp.concatenate(outs, axis=1) if heads > 1 else outs[0]


def _attention(q, k, v, heads):
    N, C = q.shape
    BQ = 400
    body = functools.partial(_attn_body, heads)
    return pl.pallas_call(
        body,
        grid=(N // BQ,),
        in_specs=[
            pl.BlockSpec((BQ, C), lambda qb: (qb, 0)),
            pl.BlockSpec((N, C), lambda qb: (0, 0)),
            pl.BlockSpec((N, C), lambda qb: (0, 0)),
        ],
        out_specs=pl.BlockSpec((BQ, C), lambda qb: (qb, 0)),
        out_shape=jax.ShapeDtypeStruct((N, C), jnp.float32),
    )(q, k, v)


def _bn_in(t, g, b, eps=1e-5):
    mu = jnp.mean(t, axis=0, keepdims=True)
    d = t - mu
    var = jnp.mean(d * d, axis=0, keepdims=True)
    return d * lax.rsqrt(var + eps) * g + b


def _post1_body(x_ref, agg_ref, attn_ref, w1, b1, w2, b2, wo, bo,
                n1g, n1b, n2g, n2b, out_ref):
    xv = x_ref[...]
    h = xv + agg_ref[0] + agg_ref[1]
    hid = jax.nn.relu(jnp.dot(h, w1[...],
                              preferred_element_type=jnp.float32) + b1[...])
    gm = jnp.dot(hid, w2[...], preferred_element_type=jnp.float32) + b2[...]
    bnh = _bn_in(gm + xv, n1g[...], n1b[...])
    ao = jnp.dot(attn_ref[...], wo[...],
                 preferred_element_type=jnp.float32) + bo[...]
    bnha = _bn_in(ao + xv, n2g[...], n2b[...])
    out_ref[...] = bnh + bnha


def _post1(x, agg, attn, p):
    N, C = x.shape
    r = lambda a: a.reshape(1, C)
    return pl.pallas_call(
        _post1_body,
        out_shape=jax.ShapeDtypeStruct((N, C), jnp.float32),
    )(x, agg, attn, p['W1'], r(p['b1']), p['W2'], r(p['b2']),
      p['Wo'], r(p['bo']), r(p['n1_g']), r(p['n1_b']),
      r(p['n2_g']), r(p['n2_b']))


def _post2_body(final, z_ref, mw1, mb1, mw2, mb2, n3g, n3b,
                bng, bnb, linw, linb, bn2g, bn2b, out_ref):
    z = z_ref[...]
    hid = jax.nn.relu(jnp.dot(z, mw1[...],
                              preferred_element_type=jnp.float32) + mb1[...])
    z2 = z + jnp.dot(hid, mw2[...],
                     preferred_element_type=jnp.float32) + mb2[...]
    z3 = _bn_in(z2, n3g[...], n3b[...])
    y = jax.nn.relu(_bn_in(z3, bng[...], bnb[...]))
    y2 = jnp.dot(y, linw[...], preferred_element_type=jnp.float32) + linb[...]
    if final:
        out_ref[...] = y2
    else:
        out_ref[...] = jax.nn.relu(_bn_in(y2, bn2g[...], bn2b[...]))


def _post2(z, gp, n_g, n_b, linW, linb, bn2_g, bn2_b, final):
    N, C = z.shape
    CO = linW.shape[1]
    if final:
        bn2_g = jnp.zeros((CO,), jnp.float32)
        bn2_b = bn2_g
    r = lambda a: a.reshape(1, -1)
    body = functools.partial(_post2_body, final)
    return pl.pallas_call(
        body,
        out_shape=jax.ShapeDtypeStruct((N, CO), jnp.float32),
    )(z, gp['mW1'], r(gp['mb1']), gp['mW2'], r(gp['mb2']),
      r(gp['n3_g']), r(gp['n3_b']), r(n_g), r(n_b),
      linW, r(linb), r(bn2_g), r(bn2_b))


def _gps_layer(x, ea, src, dst, p, heads):
    q, k, v = _qkv(x, p, heads)
    agg = _gine_agg(x, ea, src, dst)
    attn = _attention(q, k, v, heads)
    return _post1(x, agg, attn, p)


@jax.jit
def kernel(x, edge_attr, params, edge_index):
    src = edge_index[0]
    dst = edge_index[1]
    p1 = params['gps1']
    p2 = params['gps2']
    ea1, ea2 = _ea_proj(edge_attr, p1['We'], p1['be'], p2['We'], p2['be'])

    z = _gps_layer(x, ea1, src, dst, p1, 2)
    h = _post2(z, p1, params['bn1_g'], params['bn1_b'],
               params['lin1_W'], params['lin1_b'],
               params['bn2_g'], params['bn2_b'], final=False)

    z = _gps_layer(h, ea2, src, dst, p2, 1)
    out = _post2(z, p2, params['bn2_g'], params['bn2_b'],
                 params['lin2_W'], params['lin2_b'],
                 None, None, final=True)
    return out
